# baseline (speedup 1.0000x reference)
import jax
import jax.numpy as jnp
from jax.experimental import pallas as pl
from jax.experimental.pallas import tpu as pltpu

EPS_NORM = 1e-06
EPS_MEAN = 1e-06

_S = 2048
_C = 6
_L = _S * _C
_BB = 8
_CW = 384
_EXT = 512
_NCH = _L // _CW


def _sig_block(z_ref, out_ref):
    f32 = jnp.float32
    t = jax.lax.broadcasted_iota(jnp.int32, (1, _CW), 1)
    c = t % _C

    m_last_d = (t < 378).astype(f32)
    m_first_cv = (t != 0).astype(f32)
    m_last_cv = (t != 378).astype(f32)

    fd = jnp.zeros((_BB, _CW), f32)
    fp = jnp.zeros((_BB, _CW), f32)
    fr = jnp.zeros((_BB, _CW), f32)
    fc = jnp.zeros((_BB, _CW), f32)
    fn = jnp.zeros((_BB, _CW), f32)

    for j in range(_NCH):
        a = j * _CW
        w = min(_EXT, _L - a)
        ze = z_ref[:, a:a + w]
        s1 = pltpu.roll(ze, w - 1, 1)
        s6 = pltpu.roll(ze, w - 6, 1)
        s7 = pltpu.roll(s6, w - 1, 1)

        p = ze * ze + s1 * s1
        rt = jnp.sqrt(p)
        rt6 = pltpu.roll(rt, w - 6, 1)
        d = jnp.abs(s6 - ze)
        cross = ze * s7 - s1 * s6
        norms = rt * rt6
        ok = norms > EPS_NORM
        curv = jnp.where(ok, jnp.abs(cross) / jnp.where(ok, norms, 1.0), 0.0)
        cnt = jnp.where(ok, f32(1.0), f32(0.0))

        d = d[:, :_CW]
        curv = curv[:, :_CW]
        cnt = cnt[:, :_CW]
        if j == 0:
            curv = curv * m_first_cv
            cnt = cnt * m_first_cv
        if j == _NCH - 1:
            d = d * m_last_d
            curv = curv * m_last_cv
            cnt = cnt * m_last_cv
        fd = fd + d
        fp = fp + p[:, :_CW]
        fr = fr + rt[:, :_CW]
        fc = fc + curv
        fn = fn + cnt

    mvel = ((c >= 2) & (c <= 3)).astype(f32)
    macc = (c >= 4).astype(f32)
    mspd = (c == 2).astype(f32)
    mfrc = (c == 4).astype(f32)
    mcv = (c == 0).astype(f32)

    def rowsum(x):
        return jnp.sum(x, axis=1, keepdims=True)

    sd_vel = rowsum(fd * mvel)
    sd_acc = rowsum(fd * macc)
    ss2 = rowsum(fp * mspd)
    sf2 = rowsum(fp * mfrc)
    ss1 = rowsum(fr * mspd)
    sf1 = rowsum(fr * mfrc)
    scurv = rowsum(fc * mcv)
    scnt = rowsum(fn * mcv)

    pc = jnp.where(scnt > 0, scurv / jnp.maximum(scnt, 1.0), 0.0)
    vs = 1.0 / (1.0 + sd_vel * (1.0 / (2 * (_S - 1))))
    aj = sd_acc * (1.0 / (2 * (_S - 1)))

    mean_s = ss1 * (1.0 / _S)
    var_s = jnp.maximum(ss2 * (1.0 / _S) - mean_s * mean_s, 0.0)
    mr = jnp.sqrt(var_s) / (mean_s + EPS_MEAN)

    mean_f = sf1 * (1.0 / _S)
    var_f = jnp.maximum(sf2 * (1.0 / _S) - mean_f * mean_f, 0.0)
    fm = jnp.sqrt(var_f) / (mean_f + EPS_MEAN)

    out_ref[...] = jnp.concatenate([pc, vs, aj, mr, fm], axis=1)


@jax.jit
def kernel(trajectories):
    b = trajectories.shape[0]
    z = trajectories.reshape(b, _L)
    grid = (b // _BB,)
    return pl.pallas_call(
        _sig_block,
        grid=grid,
        in_specs=[pl.BlockSpec((_BB, _L), lambda i: (i, 0))],
        out_specs=pl.BlockSpec((_BB, 5), lambda i: (i, 0)),
        out_shape=jax.ShapeDtypeStruct((b, 5), jnp.float32),
        compiler_params=pltpu.CompilerParams(
            dimension_semantics=("parallel",),
        ),
    )(z)


# speedup vs baseline: 1.9588x; 1.9588x over previous
import jax
import jax.numpy as jnp
from jax.experimental import pallas as pl
from jax.experimental.pallas import tpu as pltpu

EPS_NORM = 1e-06
EPS_MEAN = 1e-06

_S = 2048
_C = 6
_BB = 8


def _roll_t(x, k):
    return pltpu.roll(x, _S - k, 2)


def _sig_block(a_ref, out_ref):
    f32 = jnp.float32
    a = a_ref[...]
    t = jax.lax.broadcasted_iota(jnp.int32, (1, 1, _S), 2)
    m_last = (t < _S - 1).astype(f32)
    m_cv = (t < _S - 2).astype(f32)

    a1 = _roll_t(a, 1)
    ac = pltpu.roll(a, _C - 1, 1)
    ac1 = _roll_t(ac, 1)

    q = a * a
    p = q + ac * ac
    rt = jnp.sqrt(p)

    ad = jnp.abs(a1 - a) * m_last

    cmw = a * ac1 - ac * a1
    cr = _roll_t(cmw, 1)
    p1 = _roll_t(p, 1)
    n2 = p1 * _roll_t(p1, 1)
    okm = (n2 > EPS_NORM * EPS_NORM).astype(f32) * m_cv
    curv = jnp.abs(cr) * jax.lax.rsqrt(jnp.maximum(n2, EPS_NORM * EPS_NORM))
    curv = curv * okm

    sd = jnp.sum(ad, axis=2)
    sp = jnp.sum(p, axis=2)
    sr = jnp.sum(rt, axis=2)
    sc = jnp.sum(curv, axis=2)
    sn = jnp.sum(okm, axis=2)

    sd_vel = sd[:, 2:3] + sd[:, 3:4]
    sd_acc = sd[:, 4:5] + sd[:, 5:6]
    ss1 = sr[:, 2:3]
    sf1 = sr[:, 4:5]
    ss2 = sp[:, 2:3]
    sf2 = sp[:, 4:5]
    scurv = sc[:, 0:1]
    scnt = sn[:, 0:1]

    pc = jnp.where(scnt > 0, scurv / jnp.maximum(scnt, 1.0), 0.0)
    vs = 1.0 / (1.0 + sd_vel * (1.0 / (2 * (_S - 1))))
    aj = sd_acc * (1.0 / (2 * (_S - 1)))

    mean_s = ss1 * (1.0 / _S)
    var_s = jnp.maximum(ss2 * (1.0 / _S) - mean_s * mean_s, 0.0)
    mr = jnp.sqrt(var_s) / (mean_s + EPS_MEAN)

    mean_f = sf1 * (1.0 / _S)
    var_f = jnp.maximum(sf2 * (1.0 / _S) - mean_f * mean_f, 0.0)
    fm = jnp.sqrt(var_f) / (mean_f + EPS_MEAN)

    out_ref[...] = jnp.concatenate([pc, vs, aj, mr, fm], axis=1)


@jax.jit
def kernel(trajectories):
    b = trajectories.shape[0]
    at = jnp.transpose(trajectories, (0, 2, 1))
    grid = (b // _BB,)
    return pl.pallas_call(
        _sig_block,
        grid=grid,
        in_specs=[pl.BlockSpec((_BB, _C, _S), lambda i: (i, 0, 0))],
        out_specs=pl.BlockSpec((_BB, 5), lambda i: (i, 0)),
        out_shape=jax.ShapeDtypeStruct((b, 5), jnp.float32),
        compiler_params=pltpu.CompilerParams(
            dimension_semantics=("parallel",),
        ),
    )(at)


# speedup vs baseline: 9.3810x; 4.7892x over previous
import jax
import jax.numpy as jnp
from jax.experimental import pallas as pl
from jax.experimental.pallas import tpu as pltpu

EPS_NORM = 1e-06
EPS_MEAN = 1e-06

_S = 2048
_C = 6
_BB = 32


def _sh(v, k):
    return pltpu.roll(v, _S - k, 1)


def _sig_block(a_ref, out_ref):
    f32 = jnp.float32
    x = a_ref[0]
    y = a_ref[1]
    vx = a_ref[2]
    vy = a_ref[3]
    ax = a_ref[4]
    ay = a_ref[5]

    t = jax.lax.broadcasted_iota(jnp.int32, (1, _S), 1)
    m_last = (t < _S - 1).astype(f32)
    m_cv = (t < _S - 2).astype(f32)

    advel = (jnp.abs(_sh(vx, 1) - vx) + jnp.abs(_sh(vy, 1) - vy)) * m_last
    adacc = (jnp.abs(_sh(ax, 1) - ax) + jnp.abs(_sh(ay, 1) - ay)) * m_last

    sp2 = vx * vx + vy * vy
    fo2 = ax * ax + ay * ay
    sp1 = jnp.sqrt(sp2)
    fo1 = jnp.sqrt(fo2)

    x1 = _sh(x, 1)
    y1 = _sh(y, 1)
    cross = x1 * _sh(y, 2) - y1 * _sh(x, 2)
    p0s = x1 * x1 + y1 * y1
    n2 = p0s * _sh(p0s, 1)
    okm = (n2 > EPS_NORM * EPS_NORM).astype(f32) * m_cv
    curv = jnp.abs(cross) * jax.lax.rsqrt(jnp.maximum(n2, EPS_NORM * EPS_NORM))
    curv = curv * okm

    def rs(v):
        return jnp.sum(v, axis=1, keepdims=True)

    sd_vel = rs(advel)
    sd_acc = rs(adacc)
    ss1 = rs(sp1)
    ss2 = rs(sp2)
    sf1 = rs(fo1)
    sf2 = rs(fo2)
    scurv = rs(curv)
    scnt = rs(okm)

    pc = jnp.where(scnt > 0, scurv / jnp.maximum(scnt, 1.0), 0.0)
    vs = 1.0 / (1.0 + sd_vel * (1.0 / (2 * (_S - 1))))
    aj = sd_acc * (1.0 / (2 * (_S - 1)))

    mean_s = ss1 * (1.0 / _S)
    var_s = jnp.maximum(ss2 * (1.0 / _S) - mean_s * mean_s, 0.0)
    mr = jnp.sqrt(var_s) / (mean_s + EPS_MEAN)

    mean_f = sf1 * (1.0 / _S)
    var_f = jnp.maximum(sf2 * (1.0 / _S) - mean_f * mean_f, 0.0)
    fm = jnp.sqrt(var_f) / (mean_f + EPS_MEAN)

    out_ref[...] = jnp.concatenate([pc, vs, aj, mr, fm], axis=1)


@jax.jit
def kernel(trajectories):
    b = trajectories.shape[0]
    at = jnp.transpose(trajectories, (2, 0, 1))
    grid = (b // _BB,)
    return pl.pallas_call(
        _sig_block,
        grid=grid,
        in_specs=[pl.BlockSpec((_C, _BB, _S), lambda i: (0, i, 0))],
        out_specs=pl.BlockSpec((_BB, 5), lambda i: (i, 0)),
        out_shape=jax.ShapeDtypeStruct((b, 5), jnp.float32),
        compiler_params=pltpu.CompilerParams(
            dimension_semantics=("arbitrary",),
        ),
    )(at)


# speedup vs baseline: 12.9986x; 1.3856x over previous
import jax
import jax.numpy as jnp
from jax.experimental import pallas as pl
from jax.experimental.pallas import tpu as pltpu

EPS_NORM = 1e-06
EPS_MEAN = 1e-06

_S = 2048
_C = 6
_BB = 64


def _sh(v, k):
    return pltpu.roll(v, _S - k, 1)


def _sig_block(a_ref, out_ref):
    f32 = jnp.float32
    a6 = a_ref[...].reshape(_C * _BB, _S)
    x = a6[0:_BB]
    y = a6[_BB:2 * _BB]
    u = a6[2 * _BB:]

    t = jax.lax.broadcasted_iota(jnp.int32, (1, _S), 1)
    m_last = (t < _S - 1).astype(f32)
    m_cv = (t < _S - 2).astype(f32)

    ones = jnp.ones((_S, 1), f32)

    def rs(v):
        return jnp.dot(v, ones, preferred_element_type=f32)

    def cheap_sqrt(v):
        return v * jax.lax.rsqrt(jnp.maximum(v, 1e-30))

    r1 = _sh(a6, 1)
    x1 = r1[0:_BB]
    y1 = r1[_BB:2 * _BB]

    du = jnp.abs(r1[2 * _BB:] - u) * m_last
    s_du = rs(du)

    q = u * u
    sp2 = q[0:_BB] + q[_BB:2 * _BB]
    fo2 = q[2 * _BB:3 * _BB] + q[3 * _BB:]
    sp1 = cheap_sqrt(sp2)
    fo1 = cheap_sqrt(fo2)

    g = x * y1 - y * x1
    cross = _sh(g, 1)
    p0s = x1 * x1 + y1 * y1
    n2 = p0s * _sh(p0s, 1)
    okm = (n2 > EPS_NORM * EPS_NORM).astype(f32) * m_cv
    curv = jnp.abs(cross) * jax.lax.rsqrt(jnp.maximum(n2, EPS_NORM * EPS_NORM))
    curv = curv * okm

    sd_vel = s_du[0:_BB] + s_du[_BB:2 * _BB]
    sd_acc = s_du[2 * _BB:3 * _BB] + s_du[3 * _BB:]
    ss1 = rs(sp1)
    ss2 = rs(sp2)
    sf1 = rs(fo1)
    sf2 = rs(fo2)
    scurv = rs(curv)
    scnt = rs(okm)

    pc = jnp.where(scnt > 0, scurv / jnp.maximum(scnt, 1.0), 0.0)
    vs = 1.0 / (1.0 + sd_vel * (1.0 / (2 * (_S - 1))))
    aj = sd_acc * (1.0 / (2 * (_S - 1)))

    mean_s = ss1 * (1.0 / _S)
    var_s = jnp.maximum(ss2 * (1.0 / _S) - mean_s * mean_s, 0.0)
    mr = jnp.sqrt(var_s) / (mean_s + EPS_MEAN)

    mean_f = sf1 * (1.0 / _S)
    var_f = jnp.maximum(sf2 * (1.0 / _S) - mean_f * mean_f, 0.0)
    fm = jnp.sqrt(var_f) / (mean_f + EPS_MEAN)

    out_ref[...] = jnp.concatenate([pc, vs, aj, mr, fm], axis=1)


@jax.jit
def kernel(trajectories):
    b = trajectories.shape[0]
    at = jnp.transpose(trajectories, (2, 0, 1))
    grid = (b // _BB,)
    return pl.pallas_call(
        _sig_block,
        grid=grid,
        in_specs=[pl.BlockSpec((_C, _BB, _S), lambda i: (0, i, 0))],
        out_specs=pl.BlockSpec((_BB, 5), lambda i: (i, 0)),
        out_shape=jax.ShapeDtypeStruct((b, 5), jnp.float32),
        compiler_params=pltpu.CompilerParams(
            dimension_semantics=("arbitrary",),
        ),
    )(at)


# speedup vs baseline: 15.3389x; 1.1800x over previous
import jax
import jax.numpy as jnp
from jax.experimental import pallas as pl
from jax.experimental.pallas import tpu as pltpu

EPS_NORM = 1e-06
EPS_MEAN = 1e-06

_S = 2048
_C = 6
_BB = 128


def _sh1(v):
    return jnp.concatenate([v[:, 1:], v[:, :1]], axis=1)


def _sig_block(a_ref, out_ref):
    f32 = jnp.float32
    a6 = a_ref[...].reshape(_C * _BB, _S)
    x = a6[0:_BB]
    y = a6[_BB:2 * _BB]
    u = a6[2 * _BB:]

    ones = jnp.ones((_S, 1), f32)

    def rs(v):
        return jnp.dot(v, ones, preferred_element_type=f32)

    def cheap_sqrt(v):
        return v * jax.lax.rsqrt(jnp.maximum(v, 1e-30))

    r1 = _sh1(a6)
    x1 = r1[0:_BB]
    y1 = r1[_BB:2 * _BB]

    du = jnp.abs(r1[2 * _BB:] - u)
    s_du = rs(du) - du[:, _S - 1:_S]

    q6 = a6 * a6
    sp2 = q6[2 * _BB:3 * _BB] + q6[3 * _BB:4 * _BB]
    fo2 = q6[4 * _BB:5 * _BB] + q6[5 * _BB:]
    sp1 = cheap_sqrt(sp2)
    fo1 = cheap_sqrt(fo2)

    cr = x * y1 - y * x1
    p0 = q6[0:_BB] + q6[_BB:2 * _BB]
    pn = p0 * _sh1(p0)
    okf = (pn > EPS_NORM * EPS_NORM).astype(f32)
    cv = jnp.abs(cr) * jax.lax.rsqrt(jnp.maximum(pn, EPS_NORM * EPS_NORM))
    cv = cv * okf

    def edge2(v):
        return v[:, 0:1] + v[:, _S - 1:_S]

    scurv = rs(cv) - edge2(cv)
    scnt = rs(okf) - edge2(okf)

    sd_vel = s_du[0:_BB] + s_du[_BB:2 * _BB]
    sd_acc = s_du[2 * _BB:3 * _BB] + s_du[3 * _BB:]
    ss1 = rs(sp1)
    ss2 = rs(sp2)
    sf1 = rs(fo1)
    sf2 = rs(fo2)

    pc = jnp.where(scnt > 0, scurv / jnp.maximum(scnt, 1.0), 0.0)
    vs = 1.0 / (1.0 + sd_vel * (1.0 / (2 * (_S - 1))))
    aj = sd_acc * (1.0 / (2 * (_S - 1)))

    mean_s = ss1 * (1.0 / _S)
    var_s = jnp.maximum(ss2 * (1.0 / _S) - mean_s * mean_s, 0.0)
    mr = jnp.sqrt(var_s) / (mean_s + EPS_MEAN)

    mean_f = sf1 * (1.0 / _S)
    var_f = jnp.maximum(sf2 * (1.0 / _S) - mean_f * mean_f, 0.0)
    fm = jnp.sqrt(var_f) / (mean_f + EPS_MEAN)

    out_ref[...] = jnp.concatenate([pc, vs, aj, mr, fm], axis=1)


@jax.jit
def kernel(trajectories):
    b = trajectories.shape[0]
    at = jnp.transpose(trajectories, (2, 0, 1))
    grid = (b // _BB,)
    return pl.pallas_call(
        _sig_block,
        grid=grid,
        in_specs=[pl.BlockSpec((_C, _BB, _S), lambda i: (0, i, 0))],
        out_specs=pl.BlockSpec((_BB, 5), lambda i: (i, 0)),
        out_shape=jax.ShapeDtypeStruct((b, 5), jnp.float32),
        compiler_params=pltpu.CompilerParams(
            dimension_semantics=("arbitrary",),
        ),
    )(at)


# speedup vs baseline: 16.2472x; 1.0592x over previous
import jax
import jax.numpy as jnp
from jax.experimental import pallas as pl
from jax.experimental.pallas import tpu as pltpu

EPS_NORM = 1e-06
EPS_MEAN = 1e-06

_S = 2048
_C = 6
_BB = 128


def _sh1(v):
    return jnp.concatenate([v[:, 1:], v[:, :1]], axis=1)


def _sig_block(a_ref, out_ref):
    f32 = jnp.float32
    a6 = a_ref[...].reshape(_C * _BB, _S)
    x = a6[0:_BB]
    y = a6[_BB:2 * _BB]
    u = a6[2 * _BB:]

    ones = jnp.ones((_S, 1), f32)

    def rs(v):
        return jnp.dot(v, ones, preferred_element_type=f32)

    def cheap_sqrt(v):
        return v * jax.lax.rsqrt(jnp.maximum(v, 1e-30))

    r1 = _sh1(a6)
    x1 = r1[0:_BB]
    y1 = r1[_BB:2 * _BB]

    du = jnp.abs(r1[2 * _BB:] - u)
    s_du = rs(du) - du[:, _S - 1:_S]

    q6 = a6 * a6
    q4 = q6[2 * _BB:].reshape(2, 2, _BB, _S)
    pf2 = (q4[:, 0] + q4[:, 1]).reshape(2 * _BB, _S)
    pf1 = cheap_sqrt(pf2)

    cr = x * y1 - y * x1
    p0 = q6[0:_BB] + q6[_BB:2 * _BB]
    pn = p0 * (x1 * x1 + y1 * y1)
    okf = (pn > EPS_NORM * EPS_NORM).astype(f32)
    cv = jnp.abs(cr) * jax.lax.rsqrt(jnp.maximum(pn, EPS_NORM * EPS_NORM))
    cv = cv * okf

    def edge2(v):
        return v[:, 0:1] + v[:, _S - 1:_S]

    scurv = rs(cv) - edge2(cv)
    scnt = rs(okf) - edge2(okf)

    sd_vel = s_du[0:_BB] + s_du[_BB:2 * _BB]
    sd_acc = s_du[2 * _BB:3 * _BB] + s_du[3 * _BB:]
    s1 = rs(pf1)
    s2 = rs(pf2)
    ss1, sf1 = s1[0:_BB], s1[_BB:]
    ss2, sf2 = s2[0:_BB], s2[_BB:]

    pc = jnp.where(scnt > 0, scurv / jnp.maximum(scnt, 1.0), 0.0)
    vs = 1.0 / (1.0 + sd_vel * (1.0 / (2 * (_S - 1))))
    aj = sd_acc * (1.0 / (2 * (_S - 1)))

    mean_s = ss1 * (1.0 / _S)
    var_s = jnp.maximum(ss2 * (1.0 / _S) - mean_s * mean_s, 0.0)
    mr = jnp.sqrt(var_s) / (mean_s + EPS_MEAN)

    mean_f = sf1 * (1.0 / _S)
    var_f = jnp.maximum(sf2 * (1.0 / _S) - mean_f * mean_f, 0.0)
    fm = jnp.sqrt(var_f) / (mean_f + EPS_MEAN)

    out_ref[...] = jnp.concatenate([pc, vs, aj, mr, fm], axis=1)


@jax.jit
def kernel(trajectories):
    b = trajectories.shape[0]
    at = jnp.transpose(trajectories, (2, 0, 1))
    grid = (b // _BB,)
    return pl.pallas_call(
        _sig_block,
        grid=grid,
        in_specs=[pl.BlockSpec((_C, _BB, _S), lambda i: (0, i, 0))],
        out_specs=pl.BlockSpec((_BB, 5), lambda i: (i, 0)),
        out_shape=jax.ShapeDtypeStruct((b, 5), jnp.float32),
        compiler_params=pltpu.CompilerParams(
            dimension_semantics=("arbitrary",),
        ),
    )(at)


# speedup vs baseline: 16.6224x; 1.0231x over previous
import jax
import jax.numpy as jnp
from jax.experimental import pallas as pl
from jax.experimental.pallas import tpu as pltpu

EPS_NORM = 1e-06
EPS_MEAN = 1e-06

_S = 2048
_C = 6
_BB = 128


def _sh1(v):
    return jnp.concatenate([v[:, 1:], v[:, :1]], axis=1)


def _sig_block(a_ref, out_ref):
    f32 = jnp.float32
    a6 = a_ref[...].reshape(_C * _BB, _S)
    x = a6[0:_BB]
    y = a6[_BB:2 * _BB]
    u = a6[2 * _BB:]

    ones = jnp.ones((_S, 1), f32)

    def rs(v):
        return jnp.dot(v, ones, preferred_element_type=f32)

    def cheap_sqrt(v):
        return v * jax.lax.rsqrt(jnp.maximum(v, 1e-30))

    w1 = _sh1(a6[0:2 * _BB])
    x1 = w1[0:_BB]
    y1 = w1[_BB:]

    u16 = u.astype(jnp.bfloat16)
    du = jnp.abs(_sh1(u16) - u16)
    s_du = (jnp.dot(du, jnp.ones((_S, 1), jnp.bfloat16),
                    preferred_element_type=f32)
            - du[:, _S - 1:_S].astype(f32))

    q6 = a6 * a6
    q4 = q6[2 * _BB:].reshape(2, 2, _BB, _S)
    pf2 = (q4[:, 0] + q4[:, 1]).reshape(2 * _BB, _S)
    pf1 = cheap_sqrt(pf2)

    cr = x * y1 - y * x1
    p0 = q6[0:_BB] + q6[_BB:2 * _BB]
    pn = p0 * (x1 * x1 + y1 * y1)
    okf = (pn > EPS_NORM * EPS_NORM).astype(f32)
    cv = jnp.abs(cr) * jax.lax.rsqrt(jnp.maximum(pn, EPS_NORM * EPS_NORM))
    cv = cv * okf

    def edge2(v):
        return v[:, 0:1] + v[:, _S - 1:_S]

    scurv = rs(cv) - edge2(cv)
    scnt = rs(okf) - edge2(okf)

    sd_vel = s_du[0:_BB] + s_du[_BB:2 * _BB]
    sd_acc = s_du[2 * _BB:3 * _BB] + s_du[3 * _BB:]
    s1 = rs(pf1)
    s2 = rs(pf2)
    ss1, sf1 = s1[0:_BB], s1[_BB:]
    ss2, sf2 = s2[0:_BB], s2[_BB:]

    pc = jnp.where(scnt > 0, scurv / jnp.maximum(scnt, 1.0), 0.0)
    vs = 1.0 / (1.0 + sd_vel * (1.0 / (2 * (_S - 1))))
    aj = sd_acc * (1.0 / (2 * (_S - 1)))

    mean_s = ss1 * (1.0 / _S)
    var_s = jnp.maximum(ss2 * (1.0 / _S) - mean_s * mean_s, 0.0)
    mr = jnp.sqrt(var_s) / (mean_s + EPS_MEAN)

    mean_f = sf1 * (1.0 / _S)
    var_f = jnp.maximum(sf2 * (1.0 / _S) - mean_f * mean_f, 0.0)
    fm = jnp.sqrt(var_f) / (mean_f + EPS_MEAN)

    out_ref[...] = jnp.concatenate([pc, vs, aj, mr, fm], axis=1)


@jax.jit
def kernel(trajectories):
    b = trajectories.shape[0]
    at = jnp.transpose(trajectories, (2, 0, 1))
    grid = (b // _BB,)
    return pl.pallas_call(
        _sig_block,
        grid=grid,
        in_specs=[pl.BlockSpec((_C, _BB, _S), lambda i: (0, i, 0))],
        out_specs=pl.BlockSpec((_BB, 5), lambda i: (i, 0)),
        out_shape=jax.ShapeDtypeStruct((b, 5), jnp.float32),
        compiler_params=pltpu.CompilerParams(
            dimension_semantics=("arbitrary",),
        ),
    )(at)


# speedup vs baseline: 17.0887x; 1.0281x over previous
import jax
import jax.numpy as jnp
from jax.experimental import pallas as pl
from jax.experimental.pallas import tpu as pltpu

EPS_NORM = 1e-06
EPS_MEAN = 1e-06

_S = 2048
_C = 6
_BB = 128


def _sh1(v):
    return jnp.concatenate([v[:, 1:], v[:, :1]], axis=1)


def _sig_block(a_ref, out_ref):
    f32 = jnp.float32
    a6 = a_ref[...].reshape(_C * _BB, _S)
    x = a6[0:_BB]
    y = a6[_BB:2 * _BB]
    u = a6[2 * _BB:]

    ones = jnp.ones((_S, 1), f32)

    def rs(v):
        return jnp.dot(v, ones, preferred_element_type=f32)

    w1 = _sh1(a6[0:2 * _BB])
    x1 = w1[0:_BB]
    y1 = w1[_BB:]

    u16 = u.astype(jnp.bfloat16)
    du = jnp.abs(_sh1(u16) - u16)
    s_du = (jnp.dot(du, jnp.ones((_S, 1), jnp.bfloat16),
                    preferred_element_type=f32)
            - du[:, _S - 1:_S].astype(f32))

    bf16 = jnp.bfloat16
    q16 = u16 * u16
    q4 = q16.reshape(2, 2, _BB, _S)
    pf2 = (q4[:, 0] + q4[:, 1]).reshape(2 * _BB, _S)
    pf1 = pf2 * jax.lax.rsqrt(jnp.maximum(pf2, bf16(1e-30)))

    cr = x * y1 - y * x1
    p0 = x * x + y * y
    pn = p0 * (x1 * x1 + y1 * y1)
    okf = (pn > EPS_NORM * EPS_NORM).astype(f32)
    cv = jnp.abs(cr) * jax.lax.rsqrt(jnp.maximum(pn, EPS_NORM * EPS_NORM))
    cv = cv * okf

    def edge2(v):
        return v[:, 0:1] + v[:, _S - 1:_S]

    scurv = rs(cv) - edge2(cv)
    scnt = rs(okf) - edge2(okf)

    sd_vel = s_du[0:_BB] + s_du[_BB:2 * _BB]
    sd_acc = s_du[2 * _BB:3 * _BB] + s_du[3 * _BB:]
    ones16 = jnp.ones((_S, 1), bf16)
    s1 = jnp.dot(pf1, ones16, preferred_element_type=f32)
    s2 = jnp.dot(pf2, ones16, preferred_element_type=f32)
    ss1, sf1 = s1[0:_BB], s1[_BB:]
    ss2, sf2 = s2[0:_BB], s2[_BB:]

    pc = jnp.where(scnt > 0, scurv / jnp.maximum(scnt, 1.0), 0.0)
    vs = 1.0 / (1.0 + sd_vel * (1.0 / (2 * (_S - 1))))
    aj = sd_acc * (1.0 / (2 * (_S - 1)))

    mean_s = ss1 * (1.0 / _S)
    var_s = jnp.maximum(ss2 * (1.0 / _S) - mean_s * mean_s, 0.0)
    mr = jnp.sqrt(var_s) / (mean_s + EPS_MEAN)

    mean_f = sf1 * (1.0 / _S)
    var_f = jnp.maximum(sf2 * (1.0 / _S) - mean_f * mean_f, 0.0)
    fm = jnp.sqrt(var_f) / (mean_f + EPS_MEAN)

    out_ref[...] = jnp.concatenate([pc, vs, aj, mr, fm], axis=1)


@jax.jit
def kernel(trajectories):
    b = trajectories.shape[0]
    at = jnp.transpose(trajectories, (2, 0, 1))
    grid = (b // _BB,)
    return pl.pallas_call(
        _sig_block,
        grid=grid,
        in_specs=[pl.BlockSpec((_C, _BB, _S), lambda i: (0, i, 0))],
        out_specs=pl.BlockSpec((_BB, 5), lambda i: (i, 0)),
        out_shape=jax.ShapeDtypeStruct((b, 5), jnp.float32),
        compiler_params=pltpu.CompilerParams(
            dimension_semantics=("arbitrary",),
        ),
    )(at)


# speedup vs baseline: 17.9380x; 1.0497x over previous
import jax
import jax.numpy as jnp
from jax.experimental import pallas as pl
from jax.experimental.pallas import tpu as pltpu

EPS_NORM = 1e-06
EPS_MEAN = 1e-06

_S = 2048
_C = 6
_BB = 128


def _sh1(v):
    return jnp.concatenate([v[:, 1:], v[:, :1]], axis=1)


def _sig_block(a_ref, out_ref):
    f32 = jnp.float32
    a6 = a_ref[...].reshape(_C * _BB, _S)
    x = a6[0:_BB]
    y = a6[_BB:2 * _BB]
    u = a6[2 * _BB:]

    ones = jnp.ones((_S, 1), f32)

    def rs(v):
        return jnp.dot(v, ones, preferred_element_type=f32)

    w1 = _sh1(a6[0:2 * _BB])
    x1 = w1[0:_BB]
    y1 = w1[_BB:]

    u16 = u.astype(jnp.bfloat16)
    du = jnp.abs(_sh1(u16) - u16)
    s_du = (jnp.dot(du, jnp.ones((_S, 1), jnp.bfloat16),
                    preferred_element_type=f32)
            - du[:, _S - 1:_S].astype(f32))

    bf16 = jnp.bfloat16
    q16 = u16 * u16
    q4 = q16.reshape(2, 2, _BB, _S)
    pf2 = (q4[:, 0] + q4[:, 1]).reshape(2 * _BB, _S)
    pf1 = pf2 * jax.lax.rsqrt(jnp.maximum(pf2, bf16(1e-30)))

    cr = x * y1 - y * x1
    p0 = x * x + y * y
    pn = p0 * (x1 * x1 + y1 * y1)
    okf = (pn > EPS_NORM * EPS_NORM).astype(f32)
    cv = jnp.abs(cr) * jax.lax.rsqrt(jnp.maximum(pn, EPS_NORM * EPS_NORM))
    cv = cv * okf

    def edge2(v):
        return v[:, 0:1] + v[:, _S - 1:_S]

    scurv = rs(cv) - edge2(cv)
    scnt = rs(okf) - edge2(okf)

    sd_vel = s_du[0:_BB] + s_du[_BB:2 * _BB]
    sd_acc = s_du[2 * _BB:3 * _BB] + s_du[3 * _BB:]
    ones16 = jnp.ones((_S, 1), bf16)
    s1 = jnp.dot(pf1, ones16, preferred_element_type=f32)
    s2 = jnp.dot(pf2, ones16, preferred_element_type=f32)
    ss1, sf1 = s1[0:_BB], s1[_BB:]
    ss2, sf2 = s2[0:_BB], s2[_BB:]

    pc = jnp.where(scnt > 0, scurv / jnp.maximum(scnt, 1.0), 0.0)
    vs = 1.0 / (1.0 + sd_vel * (1.0 / (2 * (_S - 1))))
    aj = sd_acc * (1.0 / (2 * (_S - 1)))

    mean_s = ss1 * (1.0 / _S)
    var_s = jnp.maximum(ss2 * (1.0 / _S) - mean_s * mean_s, 0.0)
    mr = jnp.sqrt(var_s) / (mean_s + EPS_MEAN)

    mean_f = sf1 * (1.0 / _S)
    var_f = jnp.maximum(sf2 * (1.0 / _S) - mean_f * mean_f, 0.0)
    fm = jnp.sqrt(var_f) / (mean_f + EPS_MEAN)

    out_ref[...] = jnp.concatenate(
        [v.reshape(1, _BB) for v in (pc, vs, aj, mr, fm)], axis=0)


@jax.jit
def kernel(trajectories):
    b = trajectories.shape[0]
    at = jnp.transpose(trajectories, (2, 0, 1))
    grid = (b // _BB,)
    return pl.pallas_call(
        _sig_block,
        grid=grid,
        in_specs=[pl.BlockSpec((_C, _BB, _S), lambda i: (0, i, 0))],
        out_specs=pl.BlockSpec((5, _BB), lambda i: (0, i)),
        out_shape=jax.ShapeDtypeStruct((5, b), jnp.float32),
        compiler_params=pltpu.CompilerParams(
            dimension_semantics=("arbitrary",),
        ),
    )(at).T
